# baseline (speedup 1.0000x reference)
import functools

import jax
import jax.numpy as jnp
from jax import lax
from jax.experimental import pallas as pl
from jax.experimental.pallas import tpu as pltpu
from jax.experimental.pallas import tpu_sc as plsc


_BN = 1024
_CH = 128
_RSPLIT = 16
_CSPLIT = 2


def _assign_body(x_ref, m_ref, idx_ref, cnt_ref):
    i = pl.program_id(0)
    x = x_ref[...]
    m = m_ref[...]
    bn = x.shape[0]
    nc = m.shape[0]

    x2 = jnp.sum(x * x, axis=1)
    m2 = jnp.sum(m * m, axis=1)
    dot = jax.lax.dot_general(
        x, m, (((1,), (1,)), ((), ())),
        preferred_element_type=jnp.float32)
    d2 = (x2[:, None] + m2[None, :]) - 2.0 * dot
    dist = jnp.sqrt(jnp.maximum(d2, 0.0))
    mn = jnp.min(dist, axis=1, keepdims=True)
    lane = jax.lax.broadcasted_iota(jnp.int32, (bn, nc), 1)
    idx = jnp.min(jnp.where(dist == mn, lane, nc), axis=1)
    idx_ref[...] = idx

    oh = (idx[:, None] == lane).astype(jnp.float32)
    pcnt = jnp.sum(oh, axis=0)

    @pl.when(i == 0)
    def _():
        cnt_ref[...] = pcnt

    @pl.when(i > 0)
    def _():
        cnt_ref[...] += pcnt


def _segsum_sc_body(x_hbm, idx_hbm, out_hbm, idx_v, x_v, acc):
    c = lax.axis_index("c")
    s = lax.axis_index("s")
    w = c * 16 + s
    rg = w // _CSPLIT
    cg = w % _CSPLIT
    n, k = x_hbm.shape
    kt = k // _CSPLIT
    rpt = n // _RSPLIT
    rbase = rg * rpt
    cbase = cg * kt
    nc_kt = acc.shape[0]

    zv = jnp.zeros((16,), jnp.float32)

    def zrow(i, _):
        acc[pl.ds(i * 16, 16)] = zv
        return 0
    lax.fori_loop(0, nc_kt // 16, zrow, 0)

    colv = [j0 + lax.iota(jnp.int32, 16) for j0 in range(0, kt, 16)]

    def chunk(t, _):
        pltpu.sync_copy(idx_hbm.at[pl.ds(rbase + t * _CH, _CH)], idx_v)
        pltpu.sync_copy(x_hbm.at[pl.ds(rbase + t * _CH, _CH), pl.ds(cbase, kt)],
                        x_v)

        def group(g, _):
            grp = idx_v[pl.ds(g * 16, 16)]
            for r in range(16):
                bc = lax.gather(
                    grp, jnp.full((16, 1), r, jnp.int32),
                    lax.GatherDimensionNumbers(
                        offset_dims=(), collapsed_slice_dims=(0,),
                        start_index_map=(0,)),
                    slice_sizes=(1,),
                    mode=lax.GatherScatterMode.PROMISE_IN_BOUNDS)
                row = g * 16 + r
                base = bc * kt
                for i, j0 in enumerate(range(0, kt, 16)):
                    vals = x_v[row, pl.ds(j0, 16)]
                    plsc.addupdate_scatter(acc, [base + colv[i]], vals)
            return 0

        lax.fori_loop(0, _CH // 16, group, 0)
        return 0

    lax.fori_loop(0, rpt // _CH, chunk, 0)

    pltpu.sync_copy(acc, out_hbm.at[cg, rg, :])


def _update_body(m_ref, part_ref, cnt_ref, out_ref):
    c = cnt_ref[...]
    s = jnp.concatenate(
        [jnp.sum(part_ref[g], axis=0) for g in range(_CSPLIT)], axis=-1)
    m = m_ref[...]
    mu_new = s / jnp.maximum(c, 1.0)[:, None]
    out_ref[...] = jnp.where(c[:, None] > 0, mu_new, m)


@jax.jit
def _one_iter(Xr, M):
    n, k = Xr.shape
    nc = M.shape[0]
    kt = k // _CSPLIT
    nb = n // _BN
    idx, counts = pl.pallas_call(
        _assign_body,
        grid=(nb,),
        in_specs=[
            pl.BlockSpec((_BN, k), lambda i: (i, 0)),
            pl.BlockSpec((nc, k), lambda i: (0, 0)),
        ],
        out_specs=[
            pl.BlockSpec((_BN,), lambda i: (i,)),
            pl.BlockSpec((nc,), lambda i: (0,)),
        ],
        out_shape=[
            jax.ShapeDtypeStruct((n,), jnp.int32),
            jax.ShapeDtypeStruct((nc,), jnp.float32),
        ],
    )(Xr, M)

    segsum = pl.kernel(
        _segsum_sc_body,
        out_type=jax.ShapeDtypeStruct((_CSPLIT, _RSPLIT, nc * kt), jnp.float32),
        mesh=plsc.VectorSubcoreMesh(core_axis_name="c", subcore_axis_name="s"),
        scratch_types=[
            pltpu.VMEM((_CH,), jnp.int32),
            pltpu.VMEM((_CH, kt), jnp.float32),
            pltpu.VMEM((nc * kt,), jnp.float32),
        ],
        compiler_params=pltpu.CompilerParams(needs_layout_passes=False),
    )
    partials = segsum(Xr, idx)
    partials = partials.reshape(_CSPLIT, _RSPLIT, nc, kt)

    return pl.pallas_call(
        _update_body,
        out_shape=jax.ShapeDtypeStruct((nc, k), jnp.float32),
    )(M, partials, counts)


def kernel(X, mu, niter):
    nc, _, k = mu.shape
    Xr = X.reshape(-1, k)
    M0 = mu[:, 0, :]
    Mf = jax.lax.fori_loop(0, niter, lambda t, M: _one_iter(Xr, M), M0)
    return Mf[:, None, :]


# speedup vs baseline: 1.2066x; 1.2066x over previous
import jax
import jax.numpy as jnp
from jax import lax
from jax.experimental import pallas as pl
from jax.experimental.pallas import tpu as pltpu
from jax.experimental.pallas import tpu_sc as plsc


_BN = 1024
_CH = 128
_RSPLIT = 16
_CSPLIT = 2


def _assign_body(x_ref, m_ref, idx_ref, cnt_ref):
    i = pl.program_id(0)
    x = x_ref[...]
    m = m_ref[...]
    bn = x.shape[0]
    nc = m.shape[0]

    x2 = jnp.sum(x * x, axis=1)
    m2 = jnp.sum(m * m, axis=1)
    dot = jax.lax.dot_general(
        x, m, (((1,), (1,)), ((), ())),
        preferred_element_type=jnp.float32)
    d2 = (x2[:, None] + m2[None, :]) - 2.0 * dot
    dist = jnp.sqrt(jnp.maximum(d2, 0.0))
    mn = jnp.min(dist, axis=1, keepdims=True)
    lane = jax.lax.broadcasted_iota(jnp.int32, (bn, nc), 1)
    idx = jnp.min(jnp.where(dist == mn, lane, nc), axis=1)
    idx_ref[...] = idx

    oh = (idx[:, None] == lane).astype(jnp.float32)
    pcnt = jnp.sum(oh, axis=0)

    @pl.when(i == 0)
    def _():
        cnt_ref[...] = pcnt

    @pl.when(i > 0)
    def _():
        cnt_ref[...] += pcnt


def _segsum_sc_body(x_hbm, z_hbm, idx_hbm, out_hbm,
                    idx_all, x_a, x_b, acc, sem_z, sem_a, sem_b):
    c = lax.axis_index("c")
    s = lax.axis_index("s")
    w = c * 16 + s
    rg = w // _CSPLIT
    cg = w % _CSPLIT
    n, k = x_hbm.shape
    kt = k // _CSPLIT
    rpt = n // _RSPLIT
    rbase = rg * rpt
    cbase = cg * kt
    nch = rpt // _CH

    hz = pltpu.async_copy(z_hbm.at[cg], acc, sem_z)
    hi = pltpu.async_copy(idx_hbm.at[pl.ds(rbase, rpt)], idx_all, sem_z)

    bufs = (x_a, x_b)
    sems = (sem_a, sem_b)
    hx = [None, None]
    hx[0] = pltpu.async_copy(
        x_hbm.at[pl.ds(rbase, _CH), pl.ds(cbase, kt)], x_a, sem_a)
    hz.wait()
    hi.wait()

    colv = [j0 + lax.iota(jnp.int32, 16) for j0 in range(0, kt, 16)]
    nblk = kt // 16

    for t in range(nch):
        b = t & 1
        if t + 1 < nch:
            hx[1 - b] = pltpu.async_copy(
                x_hbm.at[pl.ds(rbase + (t + 1) * _CH, _CH), pl.ds(cbase, kt)],
                bufs[1 - b], sems[1 - b])
        hx[b].wait()
        xs = bufs[b]

        def group(g, _):
            gb = idx_all[pl.ds(t * _CH + g * 16, 16)] * kt
            for r in range(16):
                bc = lax.gather(
                    gb, jnp.full((16, 1), r, jnp.int32),
                    lax.GatherDimensionNumbers(
                        offset_dims=(), collapsed_slice_dims=(0,),
                        start_index_map=(0,)),
                    slice_sizes=(1,),
                    mode=lax.GatherScatterMode.PROMISE_IN_BOUNDS)
                row = g * 16 + r
                for i in range(nblk):
                    vals = xs[row, pl.ds(i * 16, 16)]
                    plsc.addupdate_scatter(acc, [bc + colv[i]], vals)
            return 0

        lax.fori_loop(0, _CH // 16, group, 0)

    pltpu.sync_copy(acc, out_hbm.at[cg, rg, :])


def _update_body(m_ref, part_ref, cnt_ref, out_ref):
    c = cnt_ref[...]
    s = jnp.concatenate(
        [jnp.sum(part_ref[g], axis=0) for g in range(_CSPLIT)], axis=-1)
    m = m_ref[...]
    mu_new = s / jnp.maximum(c, 1.0)[:, None]
    out_ref[...] = jnp.where(c[:, None] > 0, mu_new, m)


@jax.jit
def _one_iter(Xr, Z, M):
    n, k = Xr.shape
    nc = M.shape[0]
    kt = k // _CSPLIT
    nb = n // _BN
    idx, counts = pl.pallas_call(
        _assign_body,
        grid=(nb,),
        in_specs=[
            pl.BlockSpec((_BN, k), lambda i: (i, 0)),
            pl.BlockSpec((nc, k), lambda i: (0, 0)),
        ],
        out_specs=[
            pl.BlockSpec((_BN,), lambda i: (i,)),
            pl.BlockSpec((nc,), lambda i: (0,)),
        ],
        out_shape=[
            jax.ShapeDtypeStruct((n,), jnp.int32),
            jax.ShapeDtypeStruct((nc,), jnp.float32),
        ],
    )(Xr, M)

    segsum = pl.kernel(
        _segsum_sc_body,
        out_type=jax.ShapeDtypeStruct((_CSPLIT, _RSPLIT, nc * kt), jnp.float32),
        mesh=plsc.VectorSubcoreMesh(core_axis_name="c", subcore_axis_name="s"),
        scratch_types=[
            pltpu.VMEM((n // _RSPLIT,), jnp.int32),
            pltpu.VMEM((_CH, kt), jnp.float32),
            pltpu.VMEM((_CH, kt), jnp.float32),
            pltpu.VMEM((nc * kt,), jnp.float32),
            pltpu.SemaphoreType.DMA,
            pltpu.SemaphoreType.DMA,
            pltpu.SemaphoreType.DMA,
        ],
        compiler_params=pltpu.CompilerParams(needs_layout_passes=False),
    )
    partials = segsum(Xr, Z, idx)
    partials = partials.reshape(_CSPLIT, _RSPLIT, nc, kt)

    return pl.pallas_call(
        _update_body,
        out_shape=jax.ShapeDtypeStruct((nc, k), jnp.float32),
    )(M, partials, counts)


def kernel(X, mu, niter):
    nc, _, k = mu.shape
    Xr = X.reshape(-1, k)
    M0 = mu[:, 0, :]
    Z = jnp.zeros((_CSPLIT, nc * (k // _CSPLIT)), jnp.float32)
    Mf = jax.lax.fori_loop(0, niter, lambda t, M: _one_iter(Xr, Z, M), M0)
    return Mf[:, None, :]


# speedup vs baseline: 1.3336x; 1.1052x over previous
import jax
import jax.numpy as jnp
from jax import lax
from jax.experimental import pallas as pl
from jax.experimental.pallas import tpu as pltpu
from jax.experimental.pallas import tpu_sc as plsc


_BN = 1024
_CH = 128
_RSPLIT = 16
_CSPLIT = 2


def _assign_body(x_ref, m_ref, idx_ref):
    x = x_ref[...]
    m = m_ref[...]
    bn = x.shape[0]
    nc = m.shape[0]

    x2 = jnp.sum(x * x, axis=1)
    m2 = jnp.sum(m * m, axis=1)
    dot = jax.lax.dot_general(
        x, m, (((1,), (1,)), ((), ())),
        preferred_element_type=jnp.float32)
    d2 = (x2[:, None] + m2[None, :]) - 2.0 * dot
    d2 = jnp.maximum(d2, 0.0)
    mn = jnp.min(d2, axis=1, keepdims=True)
    lane = jax.lax.broadcasted_iota(jnp.int32, (bn, nc), 1)
    idx_ref[...] = jnp.min(jnp.where(d2 == mn, lane, nc), axis=1)


def _segsum_sc_body(x_hbm, z_hbm, idx_hbm, out_hbm, outc_hbm,
                    idx_all, x_a, x_b, acc, acc_cnt, sem_z, sem_a, sem_b):
    c = lax.axis_index("c")
    s = lax.axis_index("s")
    w = c * 16 + s
    rg = w // _CSPLIT
    cg = w % _CSPLIT
    n, k = x_hbm.shape
    kt = k // _CSPLIT
    rpt = n // _RSPLIT
    rbase = rg * rpt
    cbase = cg * kt
    nch = rpt // _CH
    nc = acc_cnt.shape[0]

    hz = pltpu.async_copy(z_hbm.at[cg], acc, sem_z)
    hc = pltpu.async_copy(z_hbm.at[cg, pl.ds(0, nc)], acc_cnt, sem_z)
    hi = pltpu.async_copy(idx_hbm.at[pl.ds(rbase, rpt)], idx_all, sem_z)

    bufs = (x_a, x_b)
    sems = (sem_a, sem_b)
    hx = [None, None]
    hx[0] = pltpu.async_copy(
        x_hbm.at[pl.ds(rbase, _CH), pl.ds(cbase, kt)], x_a, sem_a)
    hz.wait()
    hc.wait()
    hi.wait()

    colv = [j0 + lax.iota(jnp.int32, 16) for j0 in range(0, kt, 16)]
    nblk = kt // 16
    ones = jnp.full((16,), 1.0, jnp.float32)

    for t in range(nch):
        b = t & 1
        if t + 1 < nch:
            hx[1 - b] = pltpu.async_copy(
                x_hbm.at[pl.ds(rbase + (t + 1) * _CH, _CH), pl.ds(cbase, kt)],
                bufs[1 - b], sems[1 - b])
        hx[b].wait()
        xs = bufs[b]

        def group(g, _):
            grp = idx_all[pl.ds(t * _CH + g * 16, 16)]
            plsc.addupdate_scatter(acc_cnt, [grp], ones)
            gb = grp * kt
            for r in range(16):
                bc = lax.gather(
                    gb, jnp.full((16, 1), r, jnp.int32),
                    lax.GatherDimensionNumbers(
                        offset_dims=(), collapsed_slice_dims=(0,),
                        start_index_map=(0,)),
                    slice_sizes=(1,),
                    mode=lax.GatherScatterMode.PROMISE_IN_BOUNDS)
                row = g * 16 + r
                for i in range(nblk):
                    vals = xs[row, pl.ds(i * 16, 16)]
                    plsc.addupdate_scatter(acc, [bc + colv[i]], vals)
            return 0

        lax.fori_loop(0, _CH // 16, group, 0)

    pltpu.sync_copy(acc, out_hbm.at[cg, rg, :])
    pltpu.sync_copy(acc_cnt, outc_hbm.at[cg, rg, :])


def _update_body(m_ref, part_ref, cntp_ref, out_ref):
    c = jnp.sum(cntp_ref[...], axis=(0, 1)) * (1.0 / _CSPLIT)
    s = jnp.concatenate(
        [jnp.sum(part_ref[g], axis=0) for g in range(_CSPLIT)], axis=-1)
    m = m_ref[...]
    mu_new = s / jnp.maximum(c, 1.0)[:, None]
    out_ref[...] = jnp.where(c[:, None] > 0, mu_new, m)


@jax.jit
def _one_iter(Xr, Z, M):
    n, k = Xr.shape
    nc = M.shape[0]
    kt = k // _CSPLIT
    nb = n // _BN
    idx = pl.pallas_call(
        _assign_body,
        grid=(nb,),
        in_specs=[
            pl.BlockSpec((_BN, k), lambda i: (i, 0)),
            pl.BlockSpec((nc, k), lambda i: (0, 0)),
        ],
        out_specs=pl.BlockSpec((_BN,), lambda i: (i,)),
        out_shape=jax.ShapeDtypeStruct((n,), jnp.int32),
    )(Xr, M)

    segsum = pl.kernel(
        _segsum_sc_body,
        out_type=[
            jax.ShapeDtypeStruct((_CSPLIT, _RSPLIT, nc * kt), jnp.float32),
            jax.ShapeDtypeStruct((_CSPLIT, _RSPLIT, nc), jnp.float32),
        ],
        mesh=plsc.VectorSubcoreMesh(core_axis_name="c", subcore_axis_name="s"),
        scratch_types=[
            pltpu.VMEM((n // _RSPLIT,), jnp.int32),
            pltpu.VMEM((_CH, kt), jnp.float32),
            pltpu.VMEM((_CH, kt), jnp.float32),
            pltpu.VMEM((nc * kt,), jnp.float32),
            pltpu.VMEM((nc,), jnp.float32),
            pltpu.SemaphoreType.DMA,
            pltpu.SemaphoreType.DMA,
            pltpu.SemaphoreType.DMA,
        ],
        compiler_params=pltpu.CompilerParams(needs_layout_passes=False),
    )
    partials, cntp = segsum(Xr, Z, idx)
    partials = partials.reshape(_CSPLIT, _RSPLIT, nc, kt)

    return pl.pallas_call(
        _update_body,
        out_shape=jax.ShapeDtypeStruct((nc, k), jnp.float32),
    )(M, partials, cntp)


def kernel(X, mu, niter):
    nc, _, k = mu.shape
    Xr = X.reshape(-1, k)
    M0 = mu[:, 0, :]
    Z = jnp.zeros((_CSPLIT, nc * (k // _CSPLIT)), jnp.float32)
    Mf = jax.lax.fori_loop(0, niter, lambda t, M: _one_iter(Xr, Z, M), M0)
    return Mf[:, None, :]
